# baseline (device time: 92625 ns/iter reference)
import jax
import jax.numpy as jnp
from jax import lax
from jax.experimental import pallas as pl
from jax.experimental.pallas import tpu as pltpu

ROWS = 4096
COLS = 1024
CHUNK = 256
N_CHUNKS = ROWS // CHUNK
W = 768
ABLATE_MATMUL = False


def kernel(x, dest):
    my_x = lax.axis_index("x")

    keep = (dest == my_x).astype(jnp.int32)
    n_keep = jnp.sum(keep)
    order = jnp.argsort(keep, stable=True).astype(jnp.int32)
    scal = n_keep.reshape(1).astype(jnp.int32)

    def body(scal_ref, ord_ref, x_ref, out_ref,
             buf_ref, recv_ref, send_sems, recv_sems):
        mx = lax.axis_index("x")
        my = lax.axis_index("y")
        mz = lax.axis_index("z")
        peer = (1 - mx, my, mz)

        nk = scal_ref[0]
        ns = ROWS - nk
        rb = mx * nk
        kb = mx * ns
        recv_base = (1 - mx) * nk
        lo = rb // CHUNK
        hi = (rb + ns + CHUNK - 1) // CHUNK
        klo = kb // CHUNK
        khi = (kb + nk + CHUNK - 1) // CHUNK
        rlo = recv_base // CHUNK
        rhi = (recv_base + ns + CHUNK - 1) // CHUNK

        ordv = ord_ref[:, :]
        riota = lax.broadcasted_iota(jnp.int32, (ROWS, 1), 0)
        in_send = riota < ns
        first_send = jnp.min(jnp.where(in_send, ordv, ROWS))
        last_send = jnp.max(jnp.where(in_send, ordv, 0))
        first_keep = jnp.min(jnp.where(in_send, ROWS, ordv))
        last_keep = jnp.max(jnp.where(in_send, 0, ordv))
        spad = jnp.where(in_send, ordv, last_send)
        send_ordv = jnp.where(
            riota < rb, first_send, pltpu.roll(spad, rb, 0)
        )
        kpart = pltpu.roll(ordv, ROWS - ns, 0)
        kpad = jnp.where(riota < nk, kpart, last_keep)
        keep_ordv = jnp.where(
            riota < kb, first_keep, pltpu.roll(kpad, kb, 0)
        )

        barrier_sem = pltpu.get_barrier_semaphore()
        pl.semaphore_signal(
            barrier_sem, inc=1, device_id=peer,
            device_id_type=pl.DeviceIdType.MESH,
        )
        pl.semaphore_wait(barrier_sem, 1)

        def chunk_rdma(i):
            return pltpu.make_async_remote_copy(
                src_ref=buf_ref.at[pl.ds(i * CHUNK, CHUNK), :],
                dst_ref=recv_ref.at[pl.ds(i * CHUNK, CHUNK), :],
                send_sem=send_sems.at[i],
                recv_sem=recv_sems.at[i],
                device_id=peer,
                device_id_type=pl.DeviceIdType.MESH,
            )

        def compact_chunk(c, gridv):
            ords = gridv[c * CHUNK:(c + 1) * CHUNK, :]
            m = jnp.min(ords)
            start = pl.multiple_of(
                jnp.minimum((m // 8) * 8, ROWS - W), 8
            )
            col = lax.broadcasted_iota(jnp.int32, (CHUNK, W), 1) + start
            p = (ords == col).astype(jnp.bfloat16)
            xw = x_ref[pl.ds(start, W), :]
            if ABLATE_MATMUL:
                return xw[0:CHUNK, :]
            rows = lax.dot_general(
                p, xw, (((1,), (0,)), ((), ())),
                preferred_element_type=jnp.float32,
            )
            return rows.astype(jnp.bfloat16)

        for c in range(N_CHUNKS):
            @pl.when((c >= lo) & (c < hi))
            def _(c=c):
                buf_ref[pl.ds(c * CHUNK, CHUNK), :] = compact_chunk(
                    c, send_ordv
                )
                chunk_rdma(c).start()

        for c in range(N_CHUNKS):
            @pl.when((c >= klo) & (c < khi))
            def _(c=c):
                out_ref[pl.ds(c * CHUNK, CHUNK), :] = compact_chunk(
                    c, keep_ordv
                )

        grow = lax.broadcasted_iota(jnp.int32, (CHUNK, 1), 0)
        for c in range(N_CHUNKS):
            @pl.when((c >= rlo) & (c < rhi))
            def _(c=c):
                chunk_rdma(c).wait_recv()
                rows = pl.ds(c * CHUNK, CHUNK)
                mask = ((grow + c * CHUNK) >= kb) & (
                    (grow + c * CHUNK) < kb + nk
                )
                out_ref[rows, :] = jnp.where(
                    mask, out_ref[rows, :], recv_ref[rows, :]
                )

        for c in range(N_CHUNKS):
            @pl.when((c >= lo) & (c < hi))
            def _(c=c):
                chunk_rdma(c).wait_send()

    return pl.pallas_call(
        body,
        out_shape=jax.ShapeDtypeStruct((ROWS, COLS), jnp.bfloat16),
        in_specs=[
            pl.BlockSpec(memory_space=pltpu.SMEM),
            pl.BlockSpec(memory_space=pltpu.VMEM),
            pl.BlockSpec(memory_space=pltpu.VMEM),
        ],
        out_specs=pl.BlockSpec(memory_space=pltpu.VMEM),
        scratch_shapes=[
            pltpu.VMEM((ROWS, COLS), jnp.bfloat16),
            pltpu.VMEM((ROWS, COLS), jnp.bfloat16),
            pltpu.SemaphoreType.DMA((N_CHUNKS,)),
            pltpu.SemaphoreType.DMA((N_CHUNKS,)),
        ],
        compiler_params=pltpu.CompilerParams(
            collective_id=0, vmem_limit_bytes=100 * 1024 * 1024
        ),
    )(scal, order.reshape(ROWS, 1), x.astype(jnp.bfloat16))


# device time: 84864 ns/iter; 1.0915x vs baseline; 1.0915x over previous
import jax
import jax.numpy as jnp
from jax import lax
from jax.experimental import pallas as pl
from jax.experimental.pallas import tpu as pltpu

ROWS = 4096
COLS = 1024
CHUNK = 256
N_CHUNKS = ROWS // CHUNK
W = 768


def kernel(x, dest):
    my_x = lax.axis_index("x")

    keep = (dest == my_x).astype(jnp.int32)
    n_keep = jnp.sum(keep)
    order = jnp.argsort(keep, stable=True).astype(jnp.int32)
    scal = n_keep.reshape(1).astype(jnp.int32)

    def body(scal_ref, ord_ref, x_ref, out_ref,
             buf_ref, recv_ref, send_sems, recv_sems):
        mx = lax.axis_index("x")
        my = lax.axis_index("y")
        mz = lax.axis_index("z")
        peer = (1 - mx, my, mz)

        nk = scal_ref[0]
        ns = ROWS - nk
        rb = mx * nk
        kb = mx * ns
        recv_base = (1 - mx) * nk
        lo = rb // CHUNK
        hi = (rb + ns + CHUNK - 1) // CHUNK
        klo = kb // CHUNK
        khi = (kb + nk + CHUNK - 1) // CHUNK
        rlo = recv_base // CHUNK
        rhi = (recv_base + ns + CHUNK - 1) // CHUNK

        ordv = ord_ref[:, :]
        riota = lax.broadcasted_iota(jnp.int32, (ROWS, 1), 0)
        in_send = riota < ns
        first_send = jnp.min(jnp.where(in_send, ordv, ROWS))
        last_send = jnp.max(jnp.where(in_send, ordv, 0))
        first_keep = jnp.min(jnp.where(in_send, ROWS, ordv))
        last_keep = jnp.max(jnp.where(in_send, 0, ordv))
        spad = jnp.where(in_send, ordv, last_send)
        send_ordv = jnp.where(
            riota < rb, first_send, pltpu.roll(spad, rb, 0)
        )
        kpart = pltpu.roll(ordv, ROWS - ns, 0)
        kpad = jnp.where(riota < nk, kpart, last_keep)
        keep_ordv = jnp.where(
            riota < kb, first_keep, pltpu.roll(kpad, kb, 0)
        )

        barrier_sem = pltpu.get_barrier_semaphore()
        pl.semaphore_signal(
            barrier_sem, inc=1, device_id=peer,
            device_id_type=pl.DeviceIdType.MESH,
        )
        pl.semaphore_wait(barrier_sem, 1)

        def chunk_rdma(i):
            return pltpu.make_async_remote_copy(
                src_ref=buf_ref.at[pl.ds(i * CHUNK, CHUNK), :],
                dst_ref=recv_ref.at[pl.ds(i * CHUNK, CHUNK), :],
                send_sem=send_sems.at[i],
                recv_sem=recv_sems.at[i],
                device_id=peer,
                device_id_type=pl.DeviceIdType.MESH,
            )

        def compact_chunk(c, gridv):
            ords = gridv[c * CHUNK:(c + 1) * CHUNK, :]
            m = jnp.min(ords)
            start = pl.multiple_of(
                jnp.minimum((m // 8) * 8, ROWS - W), 8
            )
            col = lax.broadcasted_iota(jnp.int32, (CHUNK, W), 1) + start
            p = (ords == col).astype(jnp.bfloat16)
            xw = x_ref[pl.ds(start, W), :].astype(jnp.bfloat16)
            rows = lax.dot_general(
                p, xw, (((1,), (0,)), ((), ())),
                preferred_element_type=jnp.float32,
            )
            return rows.astype(jnp.bfloat16)

        for c in range(N_CHUNKS):
            @pl.when((c >= lo) & (c < hi))
            def _(c=c):
                buf_ref[pl.ds(c * CHUNK, CHUNK), :] = compact_chunk(
                    c, send_ordv
                )
                chunk_rdma(c).start()

        for c in range(N_CHUNKS):
            @pl.when((c >= klo) & (c < khi))
            def _(c=c):
                out_ref[pl.ds(c * CHUNK, CHUNK), :] = compact_chunk(
                    c, keep_ordv
                )

        grow = lax.broadcasted_iota(jnp.int32, (CHUNK, 1), 0)
        for c in range(N_CHUNKS):
            @pl.when((c >= rlo) & (c < rhi))
            def _(c=c):
                chunk_rdma(c).wait_recv()
                rows = pl.ds(c * CHUNK, CHUNK)
                mask = ((grow + c * CHUNK) >= kb) & (
                    (grow + c * CHUNK) < kb + nk
                )
                out_ref[rows, :] = jnp.where(
                    mask, out_ref[rows, :], recv_ref[rows, :]
                )

        for c in range(N_CHUNKS):
            @pl.when((c >= lo) & (c < hi))
            def _(c=c):
                chunk_rdma(c).wait_send()

    return pl.pallas_call(
        body,
        out_shape=jax.ShapeDtypeStruct((ROWS, COLS), jnp.bfloat16),
        in_specs=[
            pl.BlockSpec(memory_space=pltpu.SMEM),
            pl.BlockSpec(memory_space=pltpu.VMEM),
            pl.BlockSpec(memory_space=pltpu.VMEM),
        ],
        out_specs=pl.BlockSpec(memory_space=pltpu.VMEM),
        scratch_shapes=[
            pltpu.VMEM((ROWS, COLS), jnp.bfloat16),
            pltpu.VMEM((ROWS, COLS), jnp.bfloat16),
            pltpu.SemaphoreType.DMA((N_CHUNKS,)),
            pltpu.SemaphoreType.DMA((N_CHUNKS,)),
        ],
        compiler_params=pltpu.CompilerParams(
            collective_id=0, vmem_limit_bytes=100 * 1024 * 1024
        ),
    )(scal, order.reshape(ROWS, 1), x)


# device time: 84769 ns/iter; 1.0927x vs baseline; 1.0011x over previous
import jax
import jax.numpy as jnp
from jax import lax
from jax.experimental import pallas as pl
from jax.experimental.pallas import tpu as pltpu

ROWS = 4096
COLS = 1024
CHUNK = 256
N_CHUNKS = ROWS // CHUNK
W = 768


def kernel(x, dest):
    my_x = lax.axis_index("x")

    keep = (dest == my_x).astype(jnp.int32)
    n_keep = jnp.sum(keep)
    order = jnp.argsort(keep, stable=True).astype(jnp.int32)
    scal = n_keep.reshape(1).astype(jnp.int32)

    def body(scal_ref, ord_ref, x_ref, out_ref,
             buf_ref, recv_ref, send_sems, recv_sems):
        mx = lax.axis_index("x")
        my = lax.axis_index("y")
        mz = lax.axis_index("z")
        peer = (1 - mx, my, mz)

        nk = scal_ref[0]
        ns = ROWS - nk
        rb = mx * nk
        kb = mx * ns
        recv_base = (1 - mx) * nk
        lo = rb // CHUNK
        hi = (rb + ns + CHUNK - 1) // CHUNK
        klo = kb // CHUNK
        khi = (kb + nk + CHUNK - 1) // CHUNK
        rlo = recv_base // CHUNK
        rhi = (recv_base + ns + CHUNK - 1) // CHUNK

        ordv = ord_ref[:, :]
        riota = lax.broadcasted_iota(jnp.int32, (ROWS, 1), 0)
        in_send = riota < ns
        first_send = jnp.min(jnp.where(in_send, ordv, ROWS))
        last_send = jnp.max(jnp.where(in_send, ordv, 0))
        first_keep = jnp.min(jnp.where(in_send, ROWS, ordv))
        last_keep = jnp.max(jnp.where(in_send, 0, ordv))
        spad = jnp.where(in_send, ordv, last_send)
        send_ordv = jnp.where(
            riota < rb, first_send, pltpu.roll(spad, rb, 0)
        )
        kpart = pltpu.roll(ordv, ROWS - ns, 0)
        kpad = jnp.where(riota < nk, kpart, last_keep)
        keep_ordv = jnp.where(
            riota < kb, first_keep, pltpu.roll(kpad, kb, 0)
        )

        barrier_sem = pltpu.get_barrier_semaphore()
        pl.semaphore_signal(
            barrier_sem, inc=1, device_id=peer,
            device_id_type=pl.DeviceIdType.MESH,
        )
        pl.semaphore_wait(barrier_sem, 1)

        b = ((1 - mx) * nk + mx * ns) // CHUNK
        b_send = ((1 - mx) * ns + mx * nk) // CHUNK

        def chunk_rdma(i, to_scratch):
            dst = (
                recv_ref.at[:, :]
                if to_scratch
                else out_ref.at[pl.ds(i * CHUNK, CHUNK), :]
            )
            return pltpu.make_async_remote_copy(
                src_ref=buf_ref.at[pl.ds(i * CHUNK, CHUNK), :],
                dst_ref=dst,
                send_sem=send_sems.at[i],
                recv_sem=recv_sems.at[i],
                device_id=peer,
                device_id_type=pl.DeviceIdType.MESH,
            )

        def compact_chunk(c, gridv):
            ords = gridv[c * CHUNK:(c + 1) * CHUNK, :]
            m = jnp.min(ords)
            start = pl.multiple_of(
                jnp.minimum((m // 8) * 8, ROWS - W), 8
            )
            col = lax.broadcasted_iota(jnp.int32, (CHUNK, W), 1) + start
            p = (ords == col).astype(jnp.bfloat16)
            xw = x_ref[pl.ds(start, W), :].astype(jnp.bfloat16)
            rows = lax.dot_general(
                p, xw, (((1,), (0,)), ((), ())),
                preferred_element_type=jnp.float32,
            )
            return rows.astype(jnp.bfloat16)

        for c in range(N_CHUNKS):
            @pl.when((c >= lo) & (c < hi))
            def _(c=c):
                buf_ref[pl.ds(c * CHUNK, CHUNK), :] = compact_chunk(
                    c, send_ordv
                )

                @pl.when(c == b_send)
                def _():
                    chunk_rdma(c, True).start()

                @pl.when(c != b_send)
                def _():
                    chunk_rdma(c, False).start()

        for c in range(N_CHUNKS):
            @pl.when((c >= klo) & (c < khi))
            def _(c=c):
                out_ref[pl.ds(c * CHUNK, CHUNK), :] = compact_chunk(
                    c, keep_ordv
                )

        grow = lax.broadcasted_iota(jnp.int32, (CHUNK, 1), 0)
        for c in range(N_CHUNKS):
            @pl.when((c >= rlo) & (c < rhi) & (c == b))
            def _(c=c):
                chunk_rdma(c, True).wait_recv()
                rows = pl.ds(c * CHUNK, CHUNK)
                mask = ((grow + c * CHUNK) >= kb) & (
                    (grow + c * CHUNK) < kb + nk
                )
                out_ref[rows, :] = jnp.where(
                    mask, out_ref[rows, :], recv_ref[:, :]
                )

            @pl.when((c >= rlo) & (c < rhi) & (c != b))
            def _(c=c):
                chunk_rdma(c, False).wait_recv()

        for c in range(N_CHUNKS):
            @pl.when((c >= lo) & (c < hi))
            def _(c=c):
                chunk_rdma(c, c == 0).wait_send()

    return pl.pallas_call(
        body,
        out_shape=jax.ShapeDtypeStruct((ROWS, COLS), jnp.bfloat16),
        in_specs=[
            pl.BlockSpec(memory_space=pltpu.SMEM),
            pl.BlockSpec(memory_space=pltpu.VMEM),
            pl.BlockSpec(memory_space=pltpu.VMEM),
        ],
        out_specs=pl.BlockSpec(memory_space=pltpu.VMEM),
        scratch_shapes=[
            pltpu.VMEM((ROWS, COLS), jnp.bfloat16),
            pltpu.VMEM((CHUNK, COLS), jnp.bfloat16),
            pltpu.SemaphoreType.DMA((N_CHUNKS,)),
            pltpu.SemaphoreType.DMA((N_CHUNKS,)),
        ],
        compiler_params=pltpu.CompilerParams(
            collective_id=0, vmem_limit_bytes=100 * 1024 * 1024
        ),
    )(scal, order.reshape(ROWS, 1), x)
